# baseline (speedup 1.0000x reference)
import jax
import jax.numpy as jnp
from jax.experimental import pallas as pl
from jax.experimental.pallas import tpu as pltpu

N = 100000
DIM = 128
BLK = 10000


def _body(m_ref, x_ref, o_ref):
    o_ref[...] = x_ref[...] + m_ref[0, 0]


def kernel(x, node_mask, deletion_weight):
    m = node_mask.astype(jnp.float32)[:, None]
    return pl.pallas_call(
        _body,
        grid=(N // BLK,),
        in_specs=[
            pl.BlockSpec((BLK, 1), lambda i: (i, 0)),
            pl.BlockSpec((BLK, DIM), lambda i: (i, 0)),
        ],
        out_specs=pl.BlockSpec((BLK, DIM), lambda i: (i, 0)),
        out_shape=jax.ShapeDtypeStruct((N, DIM), jnp.float32),
        compiler_params=pltpu.CompilerParams(
            dimension_semantics=("parallel",),
        ),
    )(m, x)


# speedup vs baseline: 2.3933x; 2.3933x over previous
import jax
import jax.numpy as jnp
from jax.experimental import pallas as pl
from jax.experimental.pallas import tpu as pltpu

N = 100000
DIM = 128
BLK = 10000


def _body(m_ref, w_ref, x_ref, o_ref):
    x = x_ref[...]
    m = m_ref[...].reshape(BLK, 1)
    w = w_ref[...]
    o_ref[...] = x * jnp.where(m > 0.0, w, 1.0)


def kernel(x, node_mask, deletion_weight):
    m = node_mask.astype(jnp.float32).reshape(N // BLK, 1, BLK)
    w = deletion_weight[None, :]
    return pl.pallas_call(
        _body,
        grid=(N // BLK,),
        in_specs=[
            pl.BlockSpec((1, 1, BLK), lambda i: (i, 0, 0)),
            pl.BlockSpec((1, DIM), lambda i: (0, 0)),
            pl.BlockSpec((BLK, DIM), lambda i: (i, 0)),
        ],
        out_specs=pl.BlockSpec((BLK, DIM), lambda i: (i, 0)),
        out_shape=jax.ShapeDtypeStruct((N, DIM), jnp.float32),
        compiler_params=pltpu.CompilerParams(
            dimension_semantics=("parallel",),
        ),
    )(m, w, x)
